# Initial kernel scaffold; baseline (speedup 1.0000x reference)
import jax
import jax.numpy as jnp
from jax.experimental import pallas as pl


def kernel(xyz, sup_w1, sup_b1, sup_w2, sup_b2, enh_w1, enh_b1, enh_w2, enh_b2, enh_w3, enh_b3, att_w1, att_b1, att_w2, att_b2, ft_w, ft_b, ec_w, ec_b, sp_w1, sp_b1, sp_w2, sp_b2, fu_w1, fu_b1, fu_w2, fu_b2, fu_w3, fu_b3, rg_w1, rg_b1, rg_w2, rg_b2):
    raise NotImplementedError("write your pallas kernel here")


# speedup vs baseline: 36.6160x; 36.6160x over previous
import functools

import jax
import jax.numpy as jnp
from jax import lax
from jax.experimental import pallas as pl
from jax.experimental.pallas import tpu as pltpu
from jax.experimental.pallas import tpu_sc as plsc

B, N, C, F, L, H, K, SK = 8, 2048, 3, 128, 4, 64, 16, 8
RT = 256
RT2 = 512


def _leaky(x):
    return jnp.where(x > 0, x, 0.2 * x)


def _sigmoid(x):
    return 1.0 / (1.0 + jnp.exp(-x))


def _knn_kernel(xyz_ref, xyzT_ref, ftw_ref, ftb_ref, ecw1_ref, ecw2_ref,
                ecb_ref, supw1_ref, supb1_ref, supw2_ref, supb2_ref,
                knn_ref, g_ref, cpb_ref, fw_ref):
    xr = xyz_ref[0]
    xcT = xyzT_ref[0]
    x0 = xr[:, 0:1]
    x1 = xr[:, 1:2]
    x2 = xr[:, 2:3]
    cx = xcT[0:1, :]
    cy = xcT[1:2, :]
    cz = xcT[2:3, :]
    sqc = cx * cx + cy * cy + cz * cz
    sqr = x0 * x0 + x1 * x1 + x2 * x2
    cross = jnp.dot(xr, xcT, preferred_element_type=jnp.float32)
    d = (sqr + sqc) - 2.0 * cross

    colf = lax.broadcasted_iota(jnp.int32, (RT, N), 1).astype(jnp.float32)
    big = jnp.float32(1e9)
    inf = jnp.float32(jnp.inf)
    cols = []
    for t in range(K):
        m = jnp.min(d, axis=1, keepdims=True)
        cand = jnp.where(d == m, colf, big)
        a = jnp.min(cand, axis=1, keepdims=True)
        cols.append(a)
        if t < K - 1:
            d = jnp.where(cand == a, inf, d)
    knn_ref[0] = jnp.concatenate(cols, axis=1).astype(jnp.int32)

    ftw = ftw_ref[...]
    feat = _leaky(x0 * ftw[0:1, :] + x1 * ftw[1:2, :] + x2 * ftw[2:3, :]
                  + ftb_ref[...])
    g_ref[0] = jnp.dot(feat, ecw2_ref[...], preferred_element_type=jnp.float32)
    cpb_ref[0] = (jnp.dot(feat, ecw1_ref[...], preferred_element_type=jnp.float32)
                  + ecb_ref[...])
    sw1 = supw1_ref[...]
    hw = jnp.maximum(x0 * sw1[0:1, :] + x1 * sw1[1:2, :] + x2 * sw1[2:3, :]
                     + supb1_ref[...], 0.0)
    fw_ref[0] = _sigmoid(jnp.dot(hw, supw2_ref[...],
                                 preferred_element_type=jnp.float32)
                         + supb2_ref[...])


def _run_knn(xyz, xyzT, ftw, ftb, ecw1, ecw2, ecb, supw1, supb1, supw2, supb2,
             interpret=False):
    def wspec(shape):
        return pl.BlockSpec(shape, lambda b, i: tuple(0 for _ in shape))
    grid = (B, N // RT)
    return pl.pallas_call(
        _knn_kernel,
        grid=grid,
        in_specs=[
            pl.BlockSpec((1, RT, C), lambda b, i: (b, i, 0)),
            pl.BlockSpec((1, C, N), lambda b, i: (b, 0, 0)),
            wspec((C, F)), wspec((1, F)),
            wspec((F, F)), wspec((F, F)), wspec((1, F)),
            wspec((C, H)), wspec((1, H)), wspec((H, L)), wspec((1, L)),
        ],
        out_specs=[
            pl.BlockSpec((1, RT, K), lambda b, i: (b, i, 0)),
            pl.BlockSpec((1, RT, F), lambda b, i: (b, i, 0)),
            pl.BlockSpec((1, RT, F), lambda b, i: (b, i, 0)),
            pl.BlockSpec((1, RT, L), lambda b, i: (b, i, 0)),
        ],
        out_shape=[
            jax.ShapeDtypeStruct((B, N, K), jnp.int32),
            jax.ShapeDtypeStruct((B, N, F), jnp.float32),
            jax.ShapeDtypeStruct((B, N, F), jnp.float32),
            jax.ShapeDtypeStruct((B, N, L), jnp.float32),
        ],
        interpret=interpret,
    )(xyz, xyzT, ftw, ftb, ecw1, ecw2, ecb, supw1, supb1, supw2, supb2)


def _sc_gather_stages(xyzT2, knnT4, fwT4, gflat):
    mesh = plsc.VectorSubcoreMesh(core_axis_name="c", subcore_axis_name="s",
                                  num_cores=2, num_subcores=16)
    nb = B // 2
    npt = N // 16
    nch = npt // K

    @functools.partial(
        pl.kernel,
        out_type=[
            jax.ShapeDtypeStruct((B, C * N), jnp.float32),
            jax.ShapeDtypeStruct((B, C * N), jnp.float32),
            jax.ShapeDtypeStruct((B * N, F), jnp.float32),
        ],
        mesh=mesh,
        compiler_params=pltpu.CompilerParams(needs_layout_passes=False),
        scratch_types=[
            pltpu.VMEM((C * N,), jnp.float32),
            pltpu.VMEM((K, npt), jnp.int32),
            pltpu.VMEM((L, npt), jnp.float32),
            pltpu.VMEM((C * npt,), jnp.float32),
            pltpu.VMEM((C * npt,), jnp.float32),
            pltpu.VMEM_SHARED((C * N,), jnp.float32),
            pltpu.VMEM((SK * 32,), jnp.int32),
            pltpu.VMEM((SK * 32, F), jnp.float32),
            pltpu.VMEM((32, F), jnp.float32),
            pltpu.SemaphoreType.DMA,
        ],
    )
    def sc_kernel(xyzT_hbm, knnT4_hbm, fwT4_hbm, gflat_hbm,
                  lowT_hbm, daggT_hbm, maxg_hbm,
                  LT, knn_v, fw_v, sm_v, dg_v, SH, idxbuf, rows_v, mx_v, sem):
        cid = lax.axis_index("c")
        sid = lax.axis_index("s")
        base = sid * npt
        iota16 = lax.iota(jnp.int32, 16)

        def batch_body(b_loc, _):
            b = cid * nb + b_loc
            pltpu.sync_copy(xyzT_hbm.at[b], LT)
            pltpu.sync_copy(knnT4_hbm.at[b * 16 + sid], knn_v)
            pltpu.sync_copy(fwT4_hbm.at[b * 16 + sid], fw_v)

            def zero_body(ch, _):
                for d in range(C):
                    dg_v[pl.ds(d * npt + ch * K, K)] = jnp.zeros(
                        (K,), jnp.float32)
                return 0
            lax.fori_loop(0, nch, zero_body, 0)

            for lvl in range(L):
                for d in range(C):
                    def ch_body(ch, _, d=d, lvl=lvl):
                        acc = jnp.zeros((16,), jnp.float32)
                        for kk in range(K):
                            idxv = knn_v[kk, pl.ds(ch * K, K)]
                            acc = acc + plsc.load_gather(
                                LT, [idxv + jnp.int32(d * N)])
                        sm = acc * (1.0 / K)
                        own = (base + jnp.int32(d * N)) + ch * K + iota16
                        lowv = plsc.load_gather(LT, [own])
                        det = lowv - sm
                        dg_v[pl.ds(d * npt + ch * K, K)] = (
                            dg_v[pl.ds(d * npt + ch * K, K)]
                            + fw_v[lvl, pl.ds(ch * K, K)] * det)
                        sm_v[pl.ds(d * npt + ch * K, K)] = sm
                        return 0
                    lax.fori_loop(0, nch, ch_body, 0)
                for d in range(C):
                    pltpu.sync_copy(sm_v.at[pl.ds(d * npt, npt)],
                                    SH.at[pl.ds(d * N + base, npt)])
                plsc.subcore_barrier()
                pltpu.sync_copy(SH, LT)
                plsc.subcore_barrier()

            for d in range(C):
                pltpu.sync_copy(LT.at[pl.ds(d * N + base, npt)],
                                lowT_hbm.at[b, pl.ds(d * N + base, npt)])
                pltpu.sync_copy(dg_v.at[pl.ds(d * npt, npt)],
                                daggT_hbm.at[b, pl.ds(d * N + base, npt)])

            boff = b * N

            def mchunk_body(chunk, _):
                for kk in range(SK):
                    for h in range(2):
                        src = knn_v[kk, pl.ds(chunk * 32 + h * K, K)]
                        idxbuf[pl.ds(kk * 32 + h * K, K)] = src + boff
                pltpu.async_copy(gflat_hbm.at[idxbuf], rows_v, sem).wait()

                def node_body(nd, _):
                    for cc in range(F // 16):
                        m = rows_v[nd, pl.ds(cc * 16, 16)]
                        for kk in range(1, SK):
                            m = jnp.maximum(
                                m, rows_v[kk * 32 + nd, pl.ds(cc * 16, 16)])
                        mx_v[nd, pl.ds(cc * 16, 16)] = m
                    return 0
                lax.fori_loop(0, 32, node_body, 0)
                pltpu.sync_copy(
                    mx_v, maxg_hbm.at[pl.ds(boff + base + chunk * 32, 32)])
                return 0
            lax.fori_loop(0, npt // 32, mchunk_body, 0)
            return 0

        lax.fori_loop(0, nb, batch_body, 0)

    return sc_kernel(xyzT2, knnT4, fwT4, gflat)


def _tail_kernel(low_ref, dagg_ref, maxg_ref, cpb_ref,
                 ew1_ref, eb1_ref, ew2_ref, eb2_ref, ew3T_ref, eb3_ref,
                 aw1_ref, ab1_ref, aw2_ref, ab2_ref,
                 spw1_ref, spb1_ref, spw2r_ref, spb2_ref,
                 fwl_ref, fwd_ref, fwa_ref, fb1_ref, fw2_ref, fb2_ref,
                 fw3_ref, fb3_ref, rgw1_ref, rgb1_ref, rgw2T_ref, rgb2_ref,
                 out_ref):
    lowv = low_ref[0]
    daggv = dagg_ref[0]
    l0 = lowv[:, 0:1]
    l1 = lowv[:, 1:2]
    l2 = lowv[:, 2:3]
    ew1 = ew1_ref[...]
    e = _leaky(l0 * ew1[0:1, :] + l1 * ew1[1:2, :] + l2 * ew1[2:3, :]
               + eb1_ref[...])
    e = _leaky(jnp.dot(e, ew2_ref[...], preferred_element_type=jnp.float32)
               + eb2_ref[...])
    ew3T = ew3T_ref[...]
    e3 = jnp.concatenate(
        [jnp.sum(e * ew3T[c:c + 1, :], axis=1, keepdims=True)
         for c in range(C)], axis=1) + eb3_ref[...]
    aw1 = aw1_ref[...]
    h = jnp.maximum(l0 * aw1[0:1, 0:1] + l1 * aw1[0:1, 1:2]
                    + l2 * aw1[0:1, 2:3] + ab1_ref[...], 0.0)
    att = _sigmoid(h * aw2_ref[...] + ab2_ref[...])
    elow = lowv + e3 * att

    agg0 = _leaky(cpb_ref[0] + maxg_ref[0])
    sh = _leaky(jnp.dot(agg0, spw1_ref[...],
                        preferred_element_type=jnp.float32) + spb1_ref[...])
    swt = _sigmoid(jnp.sum(sh * spw2r_ref[...], axis=1, keepdims=True)
                   + spb2_ref[...])
    agg = agg0 * swt

    e0 = elow[:, 0:1]
    e1 = elow[:, 1:2]
    e2c = elow[:, 2:3]
    d0 = daggv[:, 0:1]
    d1 = daggv[:, 1:2]
    d2c = daggv[:, 2:3]
    fwl = fwl_ref[...]
    fwd = fwd_ref[...]
    f1 = _leaky(e0 * fwl[0:1, :] + e1 * fwl[1:2, :] + e2c * fwl[2:3, :]
                + d0 * fwd[0:1, :] + d1 * fwd[1:2, :] + d2c * fwd[2:3, :]
                + jnp.dot(agg, fwa_ref[...], preferred_element_type=jnp.float32)
                + fb1_ref[...])
    f2 = _leaky(jnp.dot(f1, fw2_ref[...], preferred_element_type=jnp.float32)
                + fb2_ref[...])
    gph = (jnp.dot(f2, fw3_ref[...], preferred_element_type=jnp.float32)
           + fb3_ref[...])
    r1 = jnp.maximum(
        jnp.dot(gph, rgw1_ref[...], preferred_element_type=jnp.float32)
        + rgb1_ref[...], 0.0)
    rgw2T = rgw2T_ref[...]
    reg = jnp.tanh(jnp.concatenate(
        [jnp.sum(r1 * rgw2T[c:c + 1, :], axis=1, keepdims=True)
         for c in range(C)], axis=1) + rgb2_ref[...])
    out_ref[0] = elow + daggv + 0.1 * reg


def _run_tail(low, dagg, maxg, cpb, weights, interpret=False):
    def wspec(shape):
        return pl.BlockSpec(shape, lambda b, i: tuple(0 for _ in shape))
    grid = (B, N // RT2)
    in_specs = [
        pl.BlockSpec((1, RT2, C), lambda b, i: (b, i, 0)),
        pl.BlockSpec((1, RT2, C), lambda b, i: (b, i, 0)),
        pl.BlockSpec((1, RT2, F), lambda b, i: (b, i, 0)),
        pl.BlockSpec((1, RT2, F), lambda b, i: (b, i, 0)),
    ] + [wspec(w.shape) for w in weights]
    return pl.pallas_call(
        _tail_kernel,
        grid=grid,
        in_specs=in_specs,
        out_specs=pl.BlockSpec((1, RT2, C), lambda b, i: (b, i, 0)),
        out_shape=jax.ShapeDtypeStruct((B, N, C), jnp.float32),
        interpret=interpret,
    )(low, dagg, maxg, cpb, *weights)


def _kernel_impl(xyz, sup_w1, sup_b1, sup_w2, sup_b2, enh_w1, enh_b1, enh_w2,
                 enh_b2, enh_w3, enh_b3, att_w1, att_b1, att_w2, att_b2, ft_w,
                 ft_b, ec_w, ec_b, sp_w1, sp_b1, sp_w2, sp_b2, fu_w1, fu_b1,
                 fu_w2, fu_b2, fu_w3, fu_b3, rg_w1, rg_b1, rg_w2, rg_b2,
                 interpret=False, use_sc=True):
    f32 = jnp.float32
    xyzT = jnp.transpose(xyz, (0, 2, 1))
    knn, g, cpb, fw = _run_knn(
        xyz, xyzT, ft_w, ft_b.reshape(1, F),
        ec_w[:F, :], ec_w[F:, :], ec_b.reshape(1, F),
        sup_w1, sup_b1.reshape(1, H), sup_w2, sup_b2.reshape(1, L),
        interpret=interpret)

    if use_sc:
        knnT4 = jnp.transpose(knn, (0, 2, 1)).reshape(B, K, 16, N // 16)
        knnT4 = jnp.transpose(knnT4, (0, 2, 1, 3)).reshape(B * 16, K, N // 16)
        fwT4 = jnp.transpose(fw, (0, 2, 1)).reshape(B, L, 16, N // 16)
        fwT4 = jnp.transpose(fwT4, (0, 2, 1, 3)).reshape(B * 16, L, N // 16)
        lowTf, daggTf, maxg = _sc_gather_stages(
            xyzT.reshape(B, C * N), knnT4, fwT4, g.reshape(B * N, F))
        low = jnp.transpose(lowTf.reshape(B, C, N), (0, 2, 1))
        dagg = jnp.transpose(daggTf.reshape(B, C, N), (0, 2, 1))
        maxg = maxg.reshape(B, N, F)
    else:
        def gather(feats, idx):
            return jax.vmap(lambda f, i: f[i])(feats, idx)
        low = xyz
        dagg = jnp.zeros_like(xyz)
        for lvl in range(L):
            sm = gather(low, knn).mean(axis=2)
            dagg = dagg + fw[..., lvl:lvl + 1] * (low - sm)
            low = sm
        maxg = gather(g, knn[:, :, :SK]).max(axis=2)

    weights = [
        enh_w1, enh_b1.reshape(1, H), enh_w2, enh_b2.reshape(1, H),
        enh_w3.T, enh_b3.reshape(1, C),
        att_w1.reshape(1, C), att_b1.reshape(1, 1),
        att_w2.reshape(1, C), att_b2.reshape(1, C),
        sp_w1, sp_b1.reshape(1, F // 2),
        sp_w2.reshape(1, F // 2), sp_b2.reshape(1, 1),
        fu_w1[:C, :], fu_w1[C:2 * C, :], fu_w1[2 * C:, :],
        fu_b1.reshape(1, 256), fu_w2, fu_b2.reshape(1, 128),
        fu_w3, fu_b3.reshape(1, F),
        rg_w1, rg_b1.reshape(1, F // 2), rg_w2.T, rg_b2.reshape(1, C),
    ]
    weights = [w.astype(f32) for w in weights]
    return _run_tail(low, dagg, maxg, cpb, weights, interpret=interpret)


def kernel(xyz, sup_w1, sup_b1, sup_w2, sup_b2, enh_w1, enh_b1, enh_w2,
           enh_b2, enh_w3, enh_b3, att_w1, att_b1, att_w2, att_b2, ft_w,
           ft_b, ec_w, ec_b, sp_w1, sp_b1, sp_w2, sp_b2, fu_w1, fu_b1,
           fu_w2, fu_b2, fu_w3, fu_b3, rg_w1, rg_b1, rg_w2, rg_b2):
    return _kernel_impl(
        xyz, sup_w1, sup_b1, sup_w2, sup_b2, enh_w1, enh_b1, enh_w2, enh_b2,
        enh_w3, enh_b3, att_w1, att_b1, att_w2, att_b2, ft_w, ft_b, ec_w,
        ec_b, sp_w1, sp_b1, sp_w2, sp_b2, fu_w1, fu_b1, fu_w2, fu_b2, fu_w3,
        fu_b3, rg_w1, rg_b1, rg_w2, rg_b2)


# speedup vs baseline: 38.2374x; 1.0443x over previous
import functools

import jax
import jax.numpy as jnp
from jax import lax
from jax.experimental import pallas as pl
from jax.experimental.pallas import tpu as pltpu
from jax.experimental.pallas import tpu_sc as plsc

B, N, C, F, L, H, K, SK = 8, 2048, 3, 128, 4, 64, 16, 8
RT = 256
RT2 = 2048


def _leaky(x):
    return jnp.where(x > 0, x, 0.2 * x)


def _sigmoid(x):
    return 1.0 / (1.0 + jnp.exp(-x))


def _knn_kernel(xyz_ref, xyzT_ref, ftw_ref, ftb_ref, ecw1_ref, ecw2_ref,
                ecb_ref, supw1_ref, supb1_ref, supw2_ref, supb2_ref,
                knn_ref, g_ref, cpb_ref, fw_ref):
    xr = xyz_ref[0]
    xcT = xyzT_ref[0]
    x0 = xr[:, 0:1]
    x1 = xr[:, 1:2]
    x2 = xr[:, 2:3]
    cx = xcT[0:1, :]
    cy = xcT[1:2, :]
    cz = xcT[2:3, :]
    sqc = cx * cx + cy * cy + cz * cz
    sqr = x0 * x0 + x1 * x1 + x2 * x2
    cross = jnp.dot(xr, xcT, preferred_element_type=jnp.float32)
    d = (sqr + sqc) - 2.0 * cross

    colf = lax.broadcasted_iota(jnp.int32, (RT, N), 1).astype(jnp.float32)
    big = jnp.float32(1e9)
    inf = jnp.float32(jnp.inf)
    cols = []
    for t in range(K):
        m = jnp.min(d, axis=1, keepdims=True)
        cand = jnp.where(d == m, colf, big)
        a = jnp.min(cand, axis=1, keepdims=True)
        cols.append(a)
        if t < K - 1:
            d = jnp.where(cand == a, inf, d)
    knn_ref[0] = jnp.concatenate(cols, axis=1).astype(jnp.int32)

    ftw = ftw_ref[...]
    feat = _leaky(x0 * ftw[0:1, :] + x1 * ftw[1:2, :] + x2 * ftw[2:3, :]
                  + ftb_ref[...])
    g_ref[0] = jnp.dot(feat, ecw2_ref[...], preferred_element_type=jnp.float32)
    cpb_ref[0] = (jnp.dot(feat, ecw1_ref[...], preferred_element_type=jnp.float32)
                  + ecb_ref[...])
    sw1 = supw1_ref[...]
    hw = jnp.maximum(x0 * sw1[0:1, :] + x1 * sw1[1:2, :] + x2 * sw1[2:3, :]
                     + supb1_ref[...], 0.0)
    fw_ref[0] = _sigmoid(jnp.dot(hw, supw2_ref[...],
                                 preferred_element_type=jnp.float32)
                         + supb2_ref[...])


def _run_knn(xyz, xyzT, ftw, ftb, ecw1, ecw2, ecb, supw1, supb1, supw2, supb2,
             interpret=False):
    def wspec(shape):
        return pl.BlockSpec(shape, lambda b, i: tuple(0 for _ in shape))
    grid = (B, N // RT)
    return pl.pallas_call(
        _knn_kernel,
        grid=grid,
        in_specs=[
            pl.BlockSpec((1, RT, C), lambda b, i: (b, i, 0)),
            pl.BlockSpec((1, C, N), lambda b, i: (b, 0, 0)),
            wspec((C, F)), wspec((1, F)),
            wspec((F, F)), wspec((F, F)), wspec((1, F)),
            wspec((C, H)), wspec((1, H)), wspec((H, L)), wspec((1, L)),
        ],
        out_specs=[
            pl.BlockSpec((1, RT, K), lambda b, i: (b, i, 0)),
            pl.BlockSpec((1, RT, F), lambda b, i: (b, i, 0)),
            pl.BlockSpec((1, RT, F), lambda b, i: (b, i, 0)),
            pl.BlockSpec((1, RT, L), lambda b, i: (b, i, 0)),
        ],
        out_shape=[
            jax.ShapeDtypeStruct((B, N, K), jnp.int32),
            jax.ShapeDtypeStruct((B, N, F), jnp.float32),
            jax.ShapeDtypeStruct((B, N, F), jnp.float32),
            jax.ShapeDtypeStruct((B, N, L), jnp.float32),
        ],
        interpret=interpret,
    )(xyz, xyzT, ftw, ftb, ecw1, ecw2, ecb, supw1, supb1, supw2, supb2)


def _sc_gather_stages(xyzT2, knnT4, fwT4, gflat):
    mesh = plsc.VectorSubcoreMesh(core_axis_name="c", subcore_axis_name="s",
                                  num_cores=2, num_subcores=16)
    nb = B // 2
    npt = N // 16
    nch = npt // K

    @functools.partial(
        pl.kernel,
        out_type=[
            jax.ShapeDtypeStruct((B, C * N), jnp.float32),
            jax.ShapeDtypeStruct((B, C * N), jnp.float32),
            jax.ShapeDtypeStruct((B * N, F), jnp.float32),
        ],
        mesh=mesh,
        compiler_params=pltpu.CompilerParams(needs_layout_passes=False),
        scratch_types=[
            pltpu.VMEM((C * N,), jnp.float32),
            pltpu.VMEM((K, npt), jnp.int32),
            pltpu.VMEM((L, npt), jnp.float32),
            pltpu.VMEM((C * npt,), jnp.float32),
            pltpu.VMEM((C * npt,), jnp.float32),
            pltpu.VMEM_SHARED((C * N,), jnp.float32),
            pltpu.VMEM_SHARED((C * N,), jnp.float32),
            pltpu.VMEM((SK * 32,), jnp.int32),
            pltpu.VMEM((SK * 32,), jnp.int32),
            pltpu.VMEM((SK * 32, F), jnp.float32),
            pltpu.VMEM((SK * 32, F), jnp.float32),
            pltpu.VMEM((32, F), jnp.float32),
            pltpu.SemaphoreType.DMA,
            pltpu.SemaphoreType.DMA,
        ],
    )
    def sc_kernel(xyzT_hbm, knnT4_hbm, fwT4_hbm, gflat_hbm,
                  lowT_hbm, daggT_hbm, maxg_hbm,
                  LT, knn_v, fw_v, sm_v, dg_v, SH, SH2, idxbuf0, idxbuf1,
                  rows0, rows1, mx_v, sem0, sem1):
        cid = lax.axis_index("c")
        sid = lax.axis_index("s")
        base = sid * npt
        iota16 = lax.iota(jnp.int32, 16)

        def batch_body(b_loc, _):
            b = cid * nb + b_loc
            pltpu.sync_copy(xyzT_hbm.at[b], LT)
            pltpu.sync_copy(knnT4_hbm.at[b * 16 + sid], knn_v)
            pltpu.sync_copy(fwT4_hbm.at[b * 16 + sid], fw_v)

            def zero_body(ch, _):
                for d in range(C):
                    dg_v[pl.ds(d * npt + ch * K, K)] = jnp.zeros(
                        (K,), jnp.float32)
                return 0
            lax.fori_loop(0, nch, zero_body, 0)

            for lvl in range(L):
                for d in range(C):
                    def ch_body(ch, _, d=d, lvl=lvl):
                        acc = jnp.zeros((16,), jnp.float32)
                        for kk in range(K):
                            idxv = knn_v[kk, pl.ds(ch * K, K)]
                            acc = acc + plsc.load_gather(
                                LT, [idxv + jnp.int32(d * N)])
                        sm = acc * (1.0 / K)
                        own = (base + jnp.int32(d * N)) + ch * K + iota16
                        lowv = plsc.load_gather(LT, [own])
                        det = lowv - sm
                        dg_v[pl.ds(d * npt + ch * K, K)] = (
                            dg_v[pl.ds(d * npt + ch * K, K)]
                            + fw_v[lvl, pl.ds(ch * K, K)] * det)
                        sm_v[pl.ds(d * npt + ch * K, K)] = sm
                        return 0
                    lax.fori_loop(0, nch, ch_body, 0)
                shb = SH if lvl % 2 == 0 else SH2
                for d in range(C):
                    pltpu.sync_copy(sm_v.at[pl.ds(d * npt, npt)],
                                    shb.at[pl.ds(d * N + base, npt)])
                plsc.subcore_barrier()
                pltpu.sync_copy(shb, LT)

            for d in range(C):
                pltpu.sync_copy(LT.at[pl.ds(d * N + base, npt)],
                                lowT_hbm.at[b, pl.ds(d * N + base, npt)])
                pltpu.sync_copy(dg_v.at[pl.ds(d * npt, npt)],
                                daggT_hbm.at[b, pl.ds(d * N + base, npt)])

            boff = b * N
            nchunks = npt // 32
            bufs = [(idxbuf0, rows0, sem0), (idxbuf1, rows1, sem1)]

            def build_idx(chunk, ib):
                for kk in range(SK):
                    for h in range(2):
                        src = knn_v[kk, pl.ds(chunk * 32 + h * K, K)]
                        ib[pl.ds(kk * 32 + h * K, K)] = src + boff

            build_idx(0, idxbuf0)
            handles = [pltpu.async_copy(gflat_hbm.at[idxbuf0], rows0, sem0)]
            for chunk in range(nchunks):
                ib, rv, sm = bufs[chunk % 2]
                if chunk + 1 < nchunks:
                    nib, nrv, nsm = bufs[(chunk + 1) % 2]
                    build_idx(chunk + 1, nib)
                    handles.append(
                        pltpu.async_copy(gflat_hbm.at[nib], nrv, nsm))
                handles[chunk].wait()

                def node_body(nd, _, rv=rv):
                    for cc in range(F // 16):
                        m = rv[nd, pl.ds(cc * 16, 16)]
                        for kk in range(1, SK):
                            m = jnp.maximum(
                                m, rv[kk * 32 + nd, pl.ds(cc * 16, 16)])
                        mx_v[nd, pl.ds(cc * 16, 16)] = m
                    return 0
                lax.fori_loop(0, 32, node_body, 0)
                pltpu.sync_copy(
                    mx_v, maxg_hbm.at[pl.ds(boff + base + chunk * 32, 32)])
            return 0

        lax.fori_loop(0, nb, batch_body, 0)

    return sc_kernel(xyzT2, knnT4, fwT4, gflat)


def _tail_kernel(low_ref, dagg_ref, maxg_ref, cpb_ref,
                 ew1_ref, eb1_ref, ew2_ref, eb2_ref, ew3T_ref, eb3_ref,
                 aw1_ref, ab1_ref, aw2_ref, ab2_ref,
                 spw1_ref, spb1_ref, spw2r_ref, spb2_ref,
                 fwl_ref, fwd_ref, fwa_ref, fb1_ref, fw2_ref, fb2_ref,
                 fw3_ref, fb3_ref, rgw1_ref, rgb1_ref, rgw2T_ref, rgb2_ref,
                 out_ref):
    lowv = low_ref[0]
    daggv = dagg_ref[0]
    l0 = lowv[:, 0:1]
    l1 = lowv[:, 1:2]
    l2 = lowv[:, 2:3]
    ew1 = ew1_ref[...]
    e = _leaky(l0 * ew1[0:1, :] + l1 * ew1[1:2, :] + l2 * ew1[2:3, :]
               + eb1_ref[...])
    e = _leaky(jnp.dot(e, ew2_ref[...], preferred_element_type=jnp.float32)
               + eb2_ref[...])
    ew3T = ew3T_ref[...]
    e3 = jnp.concatenate(
        [jnp.sum(e * ew3T[c:c + 1, :], axis=1, keepdims=True)
         for c in range(C)], axis=1) + eb3_ref[...]
    aw1 = aw1_ref[...]
    h = jnp.maximum(l0 * aw1[0:1, 0:1] + l1 * aw1[0:1, 1:2]
                    + l2 * aw1[0:1, 2:3] + ab1_ref[...], 0.0)
    att = _sigmoid(h * aw2_ref[...] + ab2_ref[...])
    elow = lowv + e3 * att

    agg0 = _leaky(cpb_ref[0] + maxg_ref[0])
    sh = _leaky(jnp.dot(agg0, spw1_ref[...],
                        preferred_element_type=jnp.float32) + spb1_ref[...])
    swt = _sigmoid(jnp.sum(sh * spw2r_ref[...], axis=1, keepdims=True)
                   + spb2_ref[...])
    agg = agg0 * swt

    e0 = elow[:, 0:1]
    e1 = elow[:, 1:2]
    e2c = elow[:, 2:3]
    d0 = daggv[:, 0:1]
    d1 = daggv[:, 1:2]
    d2c = daggv[:, 2:3]
    fwl = fwl_ref[...]
    fwd = fwd_ref[...]
    f1 = _leaky(e0 * fwl[0:1, :] + e1 * fwl[1:2, :] + e2c * fwl[2:3, :]
                + d0 * fwd[0:1, :] + d1 * fwd[1:2, :] + d2c * fwd[2:3, :]
                + jnp.dot(agg, fwa_ref[...], preferred_element_type=jnp.float32)
                + fb1_ref[...])
    f2 = _leaky(jnp.dot(f1, fw2_ref[...], preferred_element_type=jnp.float32)
                + fb2_ref[...])
    gph = (jnp.dot(f2, fw3_ref[...], preferred_element_type=jnp.float32)
           + fb3_ref[...])
    r1 = jnp.maximum(
        jnp.dot(gph, rgw1_ref[...], preferred_element_type=jnp.float32)
        + rgb1_ref[...], 0.0)
    rgw2T = rgw2T_ref[...]
    reg = jnp.tanh(jnp.concatenate(
        [jnp.sum(r1 * rgw2T[c:c + 1, :], axis=1, keepdims=True)
         for c in range(C)], axis=1) + rgb2_ref[...])
    out_ref[0] = elow + daggv + 0.1 * reg


def _run_tail(low, dagg, maxg, cpb, weights, interpret=False):
    def wspec(shape):
        return pl.BlockSpec(shape, lambda b, i: tuple(0 for _ in shape))
    grid = (B, N // RT2)
    in_specs = [
        pl.BlockSpec((1, RT2, C), lambda b, i: (b, i, 0)),
        pl.BlockSpec((1, RT2, C), lambda b, i: (b, i, 0)),
        pl.BlockSpec((1, RT2, F), lambda b, i: (b, i, 0)),
        pl.BlockSpec((1, RT2, F), lambda b, i: (b, i, 0)),
    ] + [wspec(w.shape) for w in weights]
    return pl.pallas_call(
        _tail_kernel,
        grid=grid,
        in_specs=in_specs,
        out_specs=pl.BlockSpec((1, RT2, C), lambda b, i: (b, i, 0)),
        out_shape=jax.ShapeDtypeStruct((B, N, C), jnp.float32),
        interpret=interpret,
    )(low, dagg, maxg, cpb, *weights)


def _kernel_impl(xyz, sup_w1, sup_b1, sup_w2, sup_b2, enh_w1, enh_b1, enh_w2,
                 enh_b2, enh_w3, enh_b3, att_w1, att_b1, att_w2, att_b2, ft_w,
                 ft_b, ec_w, ec_b, sp_w1, sp_b1, sp_w2, sp_b2, fu_w1, fu_b1,
                 fu_w2, fu_b2, fu_w3, fu_b3, rg_w1, rg_b1, rg_w2, rg_b2,
                 interpret=False, use_sc=True):
    f32 = jnp.float32
    xyzT = jnp.transpose(xyz, (0, 2, 1))
    knn, g, cpb, fw = _run_knn(
        xyz, xyzT, ft_w, ft_b.reshape(1, F),
        ec_w[:F, :], ec_w[F:, :], ec_b.reshape(1, F),
        sup_w1, sup_b1.reshape(1, H), sup_w2, sup_b2.reshape(1, L),
        interpret=interpret)

    if use_sc:
        knnT4 = jnp.transpose(knn, (0, 2, 1)).reshape(B, K, 16, N // 16)
        knnT4 = jnp.transpose(knnT4, (0, 2, 1, 3)).reshape(B * 16, K, N // 16)
        fwT4 = jnp.transpose(fw, (0, 2, 1)).reshape(B, L, 16, N // 16)
        fwT4 = jnp.transpose(fwT4, (0, 2, 1, 3)).reshape(B * 16, L, N // 16)
        lowTf, daggTf, maxg = _sc_gather_stages(
            xyzT.reshape(B, C * N), knnT4, fwT4, g.reshape(B * N, F))
        low = jnp.transpose(lowTf.reshape(B, C, N), (0, 2, 1))
        dagg = jnp.transpose(daggTf.reshape(B, C, N), (0, 2, 1))
        maxg = maxg.reshape(B, N, F)
    else:
        def gather(feats, idx):
            return jax.vmap(lambda f, i: f[i])(feats, idx)
        low = xyz
        dagg = jnp.zeros_like(xyz)
        for lvl in range(L):
            sm = gather(low, knn).mean(axis=2)
            dagg = dagg + fw[..., lvl:lvl + 1] * (low - sm)
            low = sm
        maxg = gather(g, knn[:, :, :SK]).max(axis=2)

    weights = [
        enh_w1, enh_b1.reshape(1, H), enh_w2, enh_b2.reshape(1, H),
        enh_w3.T, enh_b3.reshape(1, C),
        att_w1.reshape(1, C), att_b1.reshape(1, 1),
        att_w2.reshape(1, C), att_b2.reshape(1, C),
        sp_w1, sp_b1.reshape(1, F // 2),
        sp_w2.reshape(1, F // 2), sp_b2.reshape(1, 1),
        fu_w1[:C, :], fu_w1[C:2 * C, :], fu_w1[2 * C:, :],
        fu_b1.reshape(1, 256), fu_w2, fu_b2.reshape(1, 128),
        fu_w3, fu_b3.reshape(1, F),
        rg_w1, rg_b1.reshape(1, F // 2), rg_w2.T, rg_b2.reshape(1, C),
    ]
    weights = [w.astype(f32) for w in weights]
    return _run_tail(low, dagg, maxg, cpb, weights, interpret=interpret)


def kernel(xyz, sup_w1, sup_b1, sup_w2, sup_b2, enh_w1, enh_b1, enh_w2,
           enh_b2, enh_w3, enh_b3, att_w1, att_b1, att_w2, att_b2, ft_w,
           ft_b, ec_w, ec_b, sp_w1, sp_b1, sp_w2, sp_b2, fu_w1, fu_b1,
           fu_w2, fu_b2, fu_w3, fu_b3, rg_w1, rg_b1, rg_w2, rg_b2):
    return _kernel_impl(
        xyz, sup_w1, sup_b1, sup_w2, sup_b2, enh_w1, enh_b1, enh_w2, enh_b2,
        enh_w3, enh_b3, att_w1, att_b1, att_w2, att_b2, ft_w, ft_b, ec_w,
        ec_b, sp_w1, sp_b1, sp_w2, sp_b2, fu_w1, fu_b1, fu_w2, fu_b2, fu_w3,
        fu_b3, rg_w1, rg_b1, rg_w2, rg_b2)
